# Initial kernel scaffold; baseline (speedup 1.0000x reference)
import jax
import jax.numpy as jnp
from jax.experimental import pallas as pl


def kernel(node_fts, data, temp1, temp2):
    raise NotImplementedError("write your pallas kernel here")


# speedup vs baseline: 29.4065x; 29.4065x over previous
import jax
import jax.numpy as jnp
from jax.experimental import pallas as pl
from jax.experimental.pallas import tpu as pltpu

SIZE = 1024
DIM = 256


def _attn_kernel(q_ref, d_ref, t_ref, o_ref):
    q = q_ref[...]
    d = d_ref[...]
    temp1 = t_ref[0, 0]
    temp2 = t_ref[0, 1]

    qn = jnp.sum(q * q, axis=1, keepdims=True)
    dn = jnp.sum(d * d, axis=1)[None, :]
    g = jax.lax.dot_general(q, d, (((1,), (1,)), ((), ())),
                            preferred_element_type=jnp.float32)
    d2 = jnp.maximum(qn + dn - 2.0 * g, 0.0)
    ds = jnp.sqrt(d2)
    s = jnp.exp(temp1 * -ds)
    m = jnp.max(s, axis=1, keepdims=True)
    e = jnp.exp(s - m)
    w = e / jnp.sum(e, axis=1, keepdims=True)
    goal = jax.lax.dot_general(w, d, (((1,), (0,)), ((), ())),
                               preferred_element_type=jnp.float32)
    lf = jax.nn.sigmoid(temp2)
    o_ref[...] = lf * goal + (1.0 - lf) * q


def kernel(node_fts, data, temp1, temp2):
    b, n, dim = node_fts.shape
    q = node_fts.reshape(b * n, dim)
    t = jnp.stack([temp1, temp2]).reshape(1, 2).astype(jnp.float32)
    out = pl.pallas_call(
        _attn_kernel,
        out_shape=jax.ShapeDtypeStruct((b * n, dim), jnp.float32),
    )(q, data, t)
    return out.reshape(b, n, dim)


# speedup vs baseline: 32.4566x; 1.1037x over previous
import jax
import jax.numpy as jnp
from jax.experimental import pallas as pl
from jax.experimental.pallas import tpu as pltpu

SIZE = 1024
DIM = 256


def _attn_kernel(q_ref, d_ref, t_ref, o_ref):
    q = q_ref[...]
    d = d_ref[...]
    temp1 = t_ref[0, 0]
    temp2 = t_ref[0, 1]

    qn = jnp.sum(q * q, axis=1, keepdims=True)
    dn = jnp.sum(d * d, axis=1)[None, :]
    g = jax.lax.dot_general(q, d, (((1,), (1,)), ((), ())),
                            preferred_element_type=jnp.float32)
    d2 = jnp.maximum(qn + dn - 2.0 * g, 1e-30)
    ds = d2 * jax.lax.rsqrt(d2)
    s = jnp.exp(temp1 * -ds)
    e = jnp.exp(s)
    r = jnp.sum(e, axis=1, keepdims=True)
    goal = jax.lax.dot_general(e, d, (((1,), (0,)), ((), ())),
                               preferred_element_type=jnp.float32)
    lf = jax.nn.sigmoid(temp2)
    o_ref[...] = (lf / r) * goal + (1.0 - lf) * q


def kernel(node_fts, data, temp1, temp2):
    b, n, dim = node_fts.shape
    q = node_fts.reshape(b * n, dim)
    t = jnp.stack([temp1, temp2]).reshape(1, 2).astype(jnp.float32)
    out = pl.pallas_call(
        _attn_kernel,
        out_shape=jax.ShapeDtypeStruct((b * n, dim), jnp.float32),
    )(q, data, t)
    return out.reshape(b, n, dim)
